# Initial kernel scaffold; baseline (speedup 1.0000x reference)
import jax
import jax.numpy as jnp
from jax.experimental import pallas as pl


def kernel(relative_positions, embedding_table):
    raise NotImplementedError("write your pallas kernel here")


# speedup vs baseline: 6.9295x; 6.9295x over previous
import functools

import jax
import jax.numpy as jnp
from jax import lax
from jax.experimental import pallas as pl
from jax.experimental.pallas import tpu as pltpu
from jax.experimental.pallas import tpu_sc as plsc

NUM_EMBEDDINGS = 512
EMBEDDING_DIM = 16
BATCH = 1
SEQ_LEN = 2048
TOTAL = BATCH * SEQ_LEN * SEQ_LEN

NUM_CORES = 2
NUM_SUBCORES = 16
NUM_WORKERS = NUM_CORES * NUM_SUBCORES
LANES = 16

PER_WORKER = TOTAL // NUM_WORKERS
CHUNK = 2048
NUM_CHUNKS = PER_WORKER // CHUNK
GATHER = 128
NUM_GATHERS = CHUNK // GATHER


_mesh = plsc.VectorSubcoreMesh(core_axis_name="c", subcore_axis_name="s")


@functools.partial(
    pl.kernel,
    mesh=_mesh,
    out_type=jax.ShapeDtypeStruct((TOTAL, EMBEDDING_DIM), jnp.float32),
    compiler_params=pltpu.CompilerParams(use_tc_tiling_on_sc=False),
    scratch_types=[
        pltpu.VMEM((CHUNK,), jnp.int32),
        pltpu.VMEM((CHUNK, EMBEDDING_DIM), jnp.float32),
        pltpu.SemaphoreType.DMA,
        pltpu.SemaphoreType.DMA,
    ],
)
def _sc_embedding_gather(table_hbm, idx_hbm, out_hbm, idx_v, rows_v,
                         idx_sem, gat_sem):
    wid = lax.axis_index("s") * NUM_CORES + lax.axis_index("c")
    base = wid * PER_WORKER

    def chunk_body(i, _):
        off = base + i * CHUNK
        pltpu.async_copy(idx_hbm.at[pl.ds(off, CHUNK)], idx_v, idx_sem).wait()

        def fix_body(j, _):
            v = idx_v[pl.ds(j * LANES, LANES)]
            v = jnp.minimum(
                jnp.maximum(v + NUM_EMBEDDINGS // 2, 0), NUM_EMBEDDINGS - 1
            )
            idx_v[pl.ds(j * LANES, LANES)] = v
            return 0

        lax.fori_loop(0, CHUNK // LANES, fix_body, 0, unroll=4)

        def fire_body(j, _):
            pltpu.async_copy(
                table_hbm.at[idx_v.at[pl.ds(j * GATHER, GATHER)]],
                rows_v.at[pl.ds(j * GATHER, GATHER)],
                gat_sem,
            )
            return 0

        lax.fori_loop(0, NUM_GATHERS, fire_body, 0)

        def drain_body(j, _):
            pltpu.make_async_copy(
                table_hbm.at[idx_v.at[pl.ds(0, GATHER)]],
                rows_v.at[pl.ds(0, GATHER)],
                gat_sem,
            ).wait()
            return 0

        lax.fori_loop(0, NUM_GATHERS, drain_body, 0)

        pltpu.sync_copy(rows_v, out_hbm.at[pl.ds(off, CHUNK)])
        return 0

    lax.fori_loop(0, NUM_CHUNKS, chunk_body, 0)


def kernel(relative_positions, embedding_table):
    idx_flat = relative_positions.reshape(TOTAL)
    out = _sc_embedding_gather(embedding_table, idx_flat)
    return out.reshape(BATCH, SEQ_LEN, SEQ_LEN, EMBEDDING_DIM)


# speedup vs baseline: 6.9407x; 1.0016x over previous
import functools

import jax
import jax.numpy as jnp
from jax import lax
from jax.experimental import pallas as pl
from jax.experimental.pallas import tpu as pltpu
from jax.experimental.pallas import tpu_sc as plsc

NUM_EMBEDDINGS = 512
EMBEDDING_DIM = 16
BATCH = 1
SEQ_LEN = 2048
TOTAL = BATCH * SEQ_LEN * SEQ_LEN

NUM_CORES = 2
NUM_SUBCORES = 16
NUM_WORKERS = NUM_CORES * NUM_SUBCORES
LANES = 16

PER_WORKER = TOTAL // NUM_WORKERS
CHUNK = 2048
NUM_CHUNKS = PER_WORKER // CHUNK
GATHER = 128
NUM_GATHERS = CHUNK // GATHER
NBUF = 2


_mesh = plsc.VectorSubcoreMesh(core_axis_name="c", subcore_axis_name="s")


@functools.partial(
    pl.kernel,
    mesh=_mesh,
    out_type=jax.ShapeDtypeStruct((TOTAL, EMBEDDING_DIM), jnp.float32),
    compiler_params=pltpu.CompilerParams(use_tc_tiling_on_sc=False),
    scratch_types=[
        pltpu.VMEM((CHUNK,), jnp.int32),
        pltpu.VMEM((CHUNK,), jnp.int32),
        pltpu.VMEM((CHUNK, EMBEDDING_DIM), jnp.float32),
        pltpu.VMEM((CHUNK, EMBEDDING_DIM), jnp.float32),
        [pltpu.SemaphoreType.DMA] * NBUF,
        pltpu.SemaphoreType.DMA,
        [pltpu.SemaphoreType.DMA] * NBUF,
    ],
)
def _sc_embedding_gather(table_hbm, idx_hbm, out_hbm, idx_v0, idx_v1,
                         rows_v0, rows_v1, idx_sems, gat_sem, wb_sems):
    wid = lax.axis_index("s") * NUM_CORES + lax.axis_index("c")
    base = wid * PER_WORKER
    idx_bufs = (idx_v0, idx_v1)
    rows_bufs = (rows_v0, rows_v1)

    def start_idx_load(i, b):
        pltpu.async_copy(
            idx_hbm.at[pl.ds(base + i * CHUNK, CHUNK)], idx_bufs[b],
            idx_sems[b],
        )

    def wait_idx(b):
        pltpu.make_async_copy(
            idx_hbm.at[pl.ds(base, CHUNK)], idx_bufs[b], idx_sems[b]
        ).wait()

    def start_writeback(i, b):
        pltpu.async_copy(
            rows_bufs[b], out_hbm.at[pl.ds(base + i * CHUNK, CHUNK)],
            wb_sems[b],
        )

    def wait_writeback(b):
        pltpu.make_async_copy(
            rows_bufs[b], out_hbm.at[pl.ds(base, CHUNK)], wb_sems[b]
        ).wait()

    start_idx_load(0, 0)
    start_idx_load(1, 1)

    def pair_body(g, _):
        for b in range(NBUF):
            i = g * NBUF + b
            idx_v = idx_bufs[b]
            rows_v = rows_bufs[b]
            wait_idx(b)

            def fix_body(j, _):
                v = idx_v[pl.ds(j * LANES, LANES)]
                v = jnp.minimum(
                    jnp.maximum(v + NUM_EMBEDDINGS // 2, 0),
                    NUM_EMBEDDINGS - 1,
                )
                idx_v[pl.ds(j * LANES, LANES)] = v
                return 0

            lax.fori_loop(0, CHUNK // LANES, fix_body, 0, unroll=4)

            @pl.when(g >= 1)
            def _():
                wait_writeback(b)

            def fire_body(j, _):
                pltpu.async_copy(
                    table_hbm.at[idx_v.at[pl.ds(j * GATHER, GATHER)]],
                    rows_v.at[pl.ds(j * GATHER, GATHER)],
                    gat_sem,
                )
                return 0

            lax.fori_loop(0, NUM_GATHERS, fire_body, 0)

            def drain_body(j, _):
                pltpu.make_async_copy(
                    table_hbm.at[idx_v.at[pl.ds(0, GATHER)]],
                    rows_v.at[pl.ds(0, GATHER)],
                    gat_sem,
                ).wait()
                return 0

            lax.fori_loop(0, NUM_GATHERS, drain_body, 0)

            @pl.when(g < NUM_CHUNKS // NBUF - 1)
            def _():
                start_idx_load(i + NBUF, b)

            start_writeback(i, b)
        return 0

    lax.fori_loop(0, NUM_CHUNKS // NBUF, pair_body, 0)
    for b in range(NBUF):
        wait_writeback(b)


def kernel(relative_positions, embedding_table):
    idx_flat = relative_positions.reshape(TOTAL)
    out = _sc_embedding_gather(embedding_table, idx_flat)
    return out.reshape(BATCH, SEQ_LEN, SEQ_LEN, EMBEDDING_DIM)


# speedup vs baseline: 11.4954x; 1.6562x over previous
import functools

import jax
import jax.numpy as jnp
from jax import lax
from jax.experimental import pallas as pl
from jax.experimental.pallas import tpu as pltpu
from jax.experimental.pallas import tpu_sc as plsc

NUM_EMBEDDINGS = 512
EMBEDDING_DIM = 16
BATCH = 1
SEQ_LEN = 2048

NUM_CORES = 2
NUM_SUBCORES = 16
NUM_WORKERS = NUM_CORES * NUM_SUBCORES
LANES = 16

ROWS_PER_WORKER = SEQ_LEN // NUM_WORKERS
GROUPS = SEQ_LEN // LANES
NBUF = 2


_mesh = plsc.VectorSubcoreMesh(core_axis_name="c", subcore_axis_name="s")


@functools.partial(
    pl.kernel,
    mesh=_mesh,
    out_type=jax.ShapeDtypeStruct(
        (BATCH, SEQ_LEN, EMBEDDING_DIM, SEQ_LEN), jnp.float32
    ),
    compiler_params=pltpu.CompilerParams(
        use_tc_tiling_on_sc=False, needs_layout_passes=False
    ),
    scratch_types=[
        pltpu.VMEM((NUM_EMBEDDINGS, EMBEDDING_DIM), jnp.float32),
        pltpu.VMEM((SEQ_LEN,), jnp.int32),
        pltpu.VMEM((SEQ_LEN,), jnp.int32),
        pltpu.VMEM((EMBEDDING_DIM, SEQ_LEN), jnp.float32),
        pltpu.VMEM((EMBEDDING_DIM, SEQ_LEN), jnp.float32),
        pltpu.SemaphoreType.DMA,
        [pltpu.SemaphoreType.DMA] * NBUF,
        [pltpu.SemaphoreType.DMA] * NBUF,
    ],
)
def _sc_embedding_gather(table_hbm, idx_hbm, out_hbm, table_v, idx_v0,
                         idx_v1, slab_v0, slab_v1, tbl_sem, idx_sems,
                         wb_sems):
    wid = lax.axis_index("s") * NUM_CORES + lax.axis_index("c")
    base_row = wid * ROWS_PER_WORKER

    idx_bufs = (idx_v0, idx_v1)
    slab_bufs = (slab_v0, slab_v1)

    def start_idx_load(i, b):
        pltpu.async_copy(
            idx_hbm.at[0, base_row + i], idx_bufs[b], idx_sems[b]
        )

    def wait_idx(b):
        pltpu.make_async_copy(
            idx_hbm.at[0, 0], idx_bufs[b], idx_sems[b]
        ).wait()

    def start_writeback(i, b):
        pltpu.async_copy(
            slab_bufs[b], out_hbm.at[0, base_row + i], wb_sems[b]
        )

    def wait_writeback(b):
        pltpu.make_async_copy(
            slab_bufs[b], out_hbm.at[0, 0], wb_sems[b]
        ).wait()

    pltpu.async_copy(table_hbm, table_v, tbl_sem).wait()
    start_idx_load(0, 0)
    start_idx_load(1, 1)

    dim_ids = [jnp.full((LANES,), d, jnp.int32) for d in range(EMBEDDING_DIM)]

    def pair_body(g, _):
        for b in range(NBUF):
            i = g * NBUF + b
            idx_v = idx_bufs[b]
            slab_v = slab_bufs[b]
            wait_idx(b)

            @pl.when(g >= 1)
            def _():
                wait_writeback(b)

            def group_body(j, _):
                v = idx_v[pl.ds(j * LANES, LANES)]
                v = jnp.minimum(
                    jnp.maximum(v + NUM_EMBEDDINGS // 2, 0),
                    NUM_EMBEDDINGS - 1,
                )
                for d in range(EMBEDDING_DIM):
                    slab_v[d, pl.ds(j * LANES, LANES)] = plsc.load_gather(
                        table_v, [v, dim_ids[d]]
                    )
                return 0

            lax.fori_loop(0, GROUPS, group_body, 0)

            @pl.when(g < ROWS_PER_WORKER // NBUF - 1)
            def _():
                start_idx_load(i + NBUF, b)

            start_writeback(i, b)
        return 0

    lax.fori_loop(0, ROWS_PER_WORKER // NBUF, pair_body, 0)
    for b in range(NBUF):
        wait_writeback(b)


def kernel(relative_positions, embedding_table):
    out_t = _sc_embedding_gather(embedding_table, relative_positions)
    return jnp.swapaxes(out_t, 2, 3)


# speedup vs baseline: 24.1756x; 2.1031x over previous
import functools

import jax
import jax.numpy as jnp
from jax import lax
from jax.experimental import pallas as pl
from jax.experimental.pallas import tpu as pltpu
from jax.experimental.pallas import tpu_sc as plsc

NUM_EMBEDDINGS = 512
EMBEDDING_DIM = 16
BATCH = 1
SEQ_LEN = 2048

NUM_CORES = 2
NUM_SUBCORES = 16
NUM_WORKERS = NUM_CORES * NUM_SUBCORES
LANES = 16

ROWS_PER_WORKER = SEQ_LEN // NUM_WORKERS
GROUPS = SEQ_LEN // LANES
NBUF = 2


_mesh = plsc.VectorSubcoreMesh(core_axis_name="c", subcore_axis_name="s")


@functools.partial(
    pl.kernel,
    mesh=_mesh,
    out_type=jax.ShapeDtypeStruct(
        (BATCH, SEQ_LEN, EMBEDDING_DIM, SEQ_LEN), jnp.float32
    ),
    compiler_params=pltpu.CompilerParams(
        use_tc_tiling_on_sc=False, needs_layout_passes=False
    ),
    scratch_types=[
        pltpu.VMEM((NUM_EMBEDDINGS, EMBEDDING_DIM), jnp.float32),
        pltpu.VMEM((SEQ_LEN,), jnp.int32),
        pltpu.VMEM((SEQ_LEN,), jnp.int32),
        pltpu.VMEM((EMBEDDING_DIM, SEQ_LEN), jnp.float32),
        pltpu.VMEM((EMBEDDING_DIM, SEQ_LEN), jnp.float32),
        pltpu.SemaphoreType.DMA,
        [pltpu.SemaphoreType.DMA] * NBUF,
        [pltpu.SemaphoreType.DMA] * NBUF,
    ],
)
def _sc_embedding_gather(table_hbm, idx_hbm, out_hbm, table_v, idx_v0,
                         idx_v1, slab_v0, slab_v1, tbl_sem, idx_sems,
                         wb_sems):
    wid = lax.axis_index("s") * NUM_CORES + lax.axis_index("c")
    base_row = wid * ROWS_PER_WORKER

    idx_bufs = (idx_v0, idx_v1)
    slab_bufs = (slab_v0, slab_v1)

    def start_idx_load(i, b):
        pltpu.async_copy(
            idx_hbm.at[0, base_row + i], idx_bufs[b], idx_sems[b]
        )

    def wait_idx(b):
        pltpu.make_async_copy(
            idx_hbm.at[0, 0], idx_bufs[b], idx_sems[b]
        ).wait()

    def start_writeback(i, b):
        pltpu.async_copy(
            slab_bufs[b], out_hbm.at[0, base_row + i], wb_sems[b]
        )

    def wait_writeback(b):
        pltpu.make_async_copy(
            slab_bufs[b], out_hbm.at[0, 0], wb_sems[b]
        ).wait()

    pltpu.async_copy(table_hbm, table_v, tbl_sem).wait()
    start_idx_load(0, 0)
    start_idx_load(1, 1)

    dim_ids = [jnp.full((LANES,), d, jnp.int32) for d in range(EMBEDDING_DIM)]

    def pair_body(g, _):
        for b in range(NBUF):
            i = g * NBUF + b
            idx_v = idx_bufs[b]
            slab_v = slab_bufs[b]
            wait_idx(b)

            @pl.when(g >= 1)
            def _():
                wait_writeback(b)

            @plsc.parallel_loop(0, GROUPS, unroll=2)
            def group_body(j):
                v = idx_v[pl.ds(j * LANES, LANES)]
                v = jnp.minimum(
                    jnp.maximum(v + NUM_EMBEDDINGS // 2, 0),
                    NUM_EMBEDDINGS - 1,
                )
                for d in range(EMBEDDING_DIM):
                    slab_v[d, pl.ds(j * LANES, LANES)] = plsc.load_gather(
                        table_v, [v, dim_ids[d]]
                    )

            @pl.when(g < ROWS_PER_WORKER // NBUF - 1)
            def _():
                start_idx_load(i + NBUF, b)

            start_writeback(i, b)
        return 0

    lax.fori_loop(0, ROWS_PER_WORKER // NBUF, pair_body, 0)
    for b in range(NBUF):
        wait_writeback(b)


def kernel(relative_positions, embedding_table):
    out_t = _sc_embedding_gather(embedding_table, relative_positions)
    return jnp.swapaxes(out_t, 2, 3)
